# Initial kernel scaffold; baseline (speedup 1.0000x reference)
import jax
import jax.numpy as jnp
from jax.experimental import pallas as pl


def kernel(x, table):
    raise NotImplementedError("write your pallas kernel here")


# speedup vs baseline: 2.1555x; 2.1555x over previous
import functools

import jax
import jax.numpy as jnp
from jax import lax
from jax.experimental import pallas as pl
from jax.experimental.pallas import tpu as pltpu
from jax.experimental.pallas import tpu_sc as plsc

D_MODEL = 64
SEQ_LEN = 200
BATCH = 4096
NUM_WORKERS = 32
CHUNK = 128
N_ROWS = BATCH * SEQ_LEN
ROWS_PER_W = N_ROWS // NUM_WORKERS
N_CHUNKS = ROWS_PER_W // CHUNK


def _make_encoding(seq_len: int) -> jax.Array:
    pos = jnp.arange(seq_len, dtype=jnp.float32)[:, None]
    _2i = jnp.arange(0, D_MODEL, 2, dtype=jnp.float32)
    enc = jnp.zeros((seq_len, D_MODEL), dtype=jnp.float32)
    enc = enc.at[:, 0::2].set(jnp.sin(pos / (10000.0 ** (_2i / D_MODEL))))
    enc = enc.at[:, 1::2].set(jnp.cos(pos / (10000.0 ** (_2i / D_MODEL))))
    return enc


def _emb_body(table_hbm, idx_hbm, enc_hbm, out_hbm,
              enc_v, idx_v, rows_v, sem):
    wid = lax.axis_index("s") * 2 + lax.axis_index("c")
    base = wid * ROWS_PER_W
    pltpu.sync_copy(enc_hbm, enc_v)

    def chunk_step(c, carry):
        row0 = base + c * CHUNK
        s_off = lax.rem(c * CHUNK, SEQ_LEN)
        pltpu.sync_copy(idx_hbm.at[pl.ds(row0, CHUNK)], idx_v)
        pltpu.async_copy(table_hbm.at[idx_v], rows_v, sem).wait()

        def add_row(j, carry2):
            for d in range(D_MODEL // 16):
                sl = pl.ds(d * 16, 16)
                rows_v[j, sl] = rows_v[j, sl] + enc_v[s_off + j, sl]
            return carry2

        lax.fori_loop(0, CHUNK, add_row, 0, unroll=2)
        pltpu.sync_copy(rows_v, out_hbm.at[pl.ds(row0, CHUNK)])
        return carry

    lax.fori_loop(0, N_CHUNKS, chunk_step, 0)


@jax.jit
def kernel(x, table):
    seq_len = x.shape[1]
    enc = _make_encoding(seq_len)
    enc2 = jnp.concatenate([enc, enc], axis=0)
    idx = x.reshape(-1)

    mesh = plsc.VectorSubcoreMesh(core_axis_name="c", subcore_axis_name="s")
    run = pl.kernel(
        _emb_body,
        out_type=jax.ShapeDtypeStruct((N_ROWS, D_MODEL), jnp.float32),
        mesh=mesh,
        scratch_types=[
            pltpu.VMEM((2 * SEQ_LEN, D_MODEL), jnp.float32),
            pltpu.VMEM((CHUNK,), jnp.int32),
            pltpu.VMEM((CHUNK, D_MODEL), jnp.float32),
            pltpu.SemaphoreType.DMA,
        ],
        compiler_params=pltpu.CompilerParams(use_tc_tiling_on_sc=False),
    )
    out = run(table, idx, enc2)
    return out.reshape(x.shape[0], seq_len, D_MODEL)


# speedup vs baseline: 2.6572x; 1.2327x over previous
import jax
import jax.numpy as jnp
from jax import lax
from jax.experimental import pallas as pl
from jax.experimental.pallas import tpu as pltpu
from jax.experimental.pallas import tpu_sc as plsc

D_MODEL = 64
SEQ_LEN = 200
BATCH = 4096
NUM_WORKERS = 32
CHUNK = 128
N_ROWS = BATCH * SEQ_LEN
ROWS_PER_W = N_ROWS // NUM_WORKERS
N_CHUNKS = ROWS_PER_W // CHUNK


def _make_encoding(seq_len: int) -> jax.Array:
    pos = jnp.arange(seq_len, dtype=jnp.float32)[:, None]
    _2i = jnp.arange(0, D_MODEL, 2, dtype=jnp.float32)
    enc = jnp.zeros((seq_len, D_MODEL), dtype=jnp.float32)
    enc = enc.at[:, 0::2].set(jnp.sin(pos / (10000.0 ** (_2i / D_MODEL))))
    enc = enc.at[:, 1::2].set(jnp.cos(pos / (10000.0 ** (_2i / D_MODEL))))
    return enc


def _emb_body(table_hbm, idx_hbm, enc_hbm, out_hbm,
              enc_v, idx_v, rows0, rows1, sg0, sg1, so0, so1, si0, si1):
    wid = lax.axis_index("s") * 2 + lax.axis_index("c")
    base = wid * ROWS_PER_W
    rows = (rows0, rows1)
    sg = (sg0, sg1)
    so = (so0, so1)
    si = (si0, si1)

    def idx_start(c, b):
        pltpu.make_async_copy(
            idx_hbm.at[pl.ds(base + c * CHUNK, CHUNK)], idx_v.at[b], si[b]
        ).start()

    def idx_wait(b):
        pltpu.make_async_copy(
            idx_hbm.at[pl.ds(base, CHUNK)], idx_v.at[b], si[b]
        ).wait()

    def gather_start(b):
        pltpu.make_async_copy(table_hbm.at[idx_v.at[b]], rows[b], sg[b]).start()

    def gather_wait(b):
        pltpu.make_async_copy(table_hbm.at[idx_v.at[b]], rows[b], sg[b]).wait()

    def store_start(c, b):
        pltpu.make_async_copy(
            rows[b], out_hbm.at[pl.ds(base + c * CHUNK, CHUNK)], so[b]
        ).start()

    def store_wait(b):
        pltpu.make_async_copy(
            rows[b], out_hbm.at[pl.ds(base, CHUNK)], so[b]
        ).wait()

    def add_enc(c, b):
        s_off = lax.rem(c * CHUNK, SEQ_LEN)
        rb = rows[b]

        def add_row(j, carry):
            for d in range(D_MODEL // 16):
                sl = pl.ds(d * 16, 16)
                rb[j, sl] = rb[j, sl] + enc_v[s_off + j, sl]
            return carry

        lax.fori_loop(0, CHUNK, add_row, 0, unroll=4)

    pltpu.sync_copy(enc_hbm, enc_v)

    pltpu.sync_copy(idx_hbm.at[pl.ds(base, CHUNK)], idx_v.at[0])
    gather_start(0)
    idx_start(1, 1)

    def pair_step(g, carry):
        for b in range(2):
            o = 1 - b
            c = 2 * g + b
            if b == 0:
                @pl.when(g >= 1)
                def _():
                    store_wait(o)
                idx_wait(o)
                gather_start(o)
            else:
                store_wait(o)

                @pl.when(g < N_CHUNKS // 2 - 1)
                def _():
                    idx_wait(o)
                    gather_start(o)
            gather_wait(b)
            add_enc(c, b)
            store_start(c, b)
            if b == 0:
                @pl.when(g < N_CHUNKS // 2 - 1)
                def _():
                    idx_start(c + 2, b)
            else:
                @pl.when(g < N_CHUNKS // 2 - 1)
                def _():
                    idx_start(c + 2, b)
        return carry

    lax.fori_loop(0, N_CHUNKS // 2, pair_step, 0)
    store_wait(1)


@jax.jit
def kernel(x, table):
    seq_len = x.shape[1]
    enc = _make_encoding(seq_len)
    enc2 = jnp.concatenate([enc, enc], axis=0)
    idx = x.reshape(-1)

    mesh = plsc.VectorSubcoreMesh(core_axis_name="c", subcore_axis_name="s")
    run = pl.kernel(
        _emb_body,
        out_type=jax.ShapeDtypeStruct((N_ROWS, D_MODEL), jnp.float32),
        mesh=mesh,
        scratch_types=[
            pltpu.VMEM((2 * SEQ_LEN, D_MODEL), jnp.float32),
            pltpu.VMEM((2, CHUNK), jnp.int32),
            pltpu.VMEM((CHUNK, D_MODEL), jnp.float32),
            pltpu.VMEM((CHUNK, D_MODEL), jnp.float32),
            pltpu.SemaphoreType.DMA,
            pltpu.SemaphoreType.DMA,
            pltpu.SemaphoreType.DMA,
            pltpu.SemaphoreType.DMA,
            pltpu.SemaphoreType.DMA,
            pltpu.SemaphoreType.DMA,
        ],
        compiler_params=pltpu.CompilerParams(use_tc_tiling_on_sc=False),
    )
    out = run(table, idx, enc2)
    return out.reshape(x.shape[0], seq_len, D_MODEL)


# speedup vs baseline: 3.7339x; 1.4052x over previous
import jax
import jax.numpy as jnp
from jax import lax
from jax.experimental import pallas as pl
from jax.experimental.pallas import tpu as pltpu
from jax.experimental.pallas import tpu_sc as plsc

D_MODEL = 64
SEQ_LEN = 200
BATCH = 4096
NUM_WORKERS = 32
CHUNK = 128
N_ROWS = BATCH * SEQ_LEN
ROWS_PER_W = N_ROWS // NUM_WORKERS
N_CHUNKS = ROWS_PER_W // CHUNK


def _make_encoding(seq_len: int) -> jax.Array:
    pos = jnp.arange(seq_len, dtype=jnp.float32)[:, None]
    _2i = jnp.arange(0, D_MODEL, 2, dtype=jnp.float32)
    enc = jnp.zeros((seq_len, D_MODEL), dtype=jnp.float32)
    enc = enc.at[:, 0::2].set(jnp.sin(pos / (10000.0 ** (_2i / D_MODEL))))
    enc = enc.at[:, 1::2].set(jnp.cos(pos / (10000.0 ** (_2i / D_MODEL))))
    return enc


def _emb_body(table_hbm, idx_hbm, enc_hbm, out_hbm,
              enc_v, idx_v, rin0, rin1, rout0, rout1,
              sg0, sg1, so0, so1, si0, si1):
    wid = lax.axis_index("s") * 2 + lax.axis_index("c")
    base = wid * ROWS_PER_W
    rin = (rin0, rin1)
    rout = (rout0, rout1)
    sg = (sg0, sg1)
    so = (so0, so1)
    si = (si0, si1)

    def idx_start(c, b):
        pltpu.make_async_copy(
            idx_hbm.at[pl.ds(base + c * CHUNK, CHUNK)], idx_v.at[b], si[b]
        ).start()

    def idx_wait(b):
        pltpu.make_async_copy(
            idx_hbm.at[pl.ds(base, CHUNK)], idx_v.at[b], si[b]
        ).wait()

    def gather_start(b):
        pltpu.make_async_copy(table_hbm.at[idx_v.at[b]], rin[b], sg[b]).start()

    def gather_wait(b):
        pltpu.make_async_copy(table_hbm.at[idx_v.at[b]], rin[b], sg[b]).wait()

    def store_start(c, b):
        pltpu.make_async_copy(
            rout[b], out_hbm.at[pl.ds(base + c * CHUNK, CHUNK)], so[b]
        ).start()

    def store_wait(b):
        pltpu.make_async_copy(
            rout[b], out_hbm.at[pl.ds(base, CHUNK)], so[b]
        ).wait()

    def add_enc(c, b):
        s_off = lax.rem(c * CHUNK, SEQ_LEN)
        src = rin[b]
        dst = rout[b]

        @plsc.parallel_loop(0, CHUNK, 1, unroll=4)
        def _(j):
            for d in range(D_MODEL // 16):
                sl = pl.ds(d * 16, 16)
                dst[j, sl] = src[j, sl] + enc_v[s_off + j, sl]

    pltpu.sync_copy(enc_hbm, enc_v)

    pltpu.sync_copy(idx_hbm.at[pl.ds(base, CHUNK)], idx_v.at[0])
    gather_start(0)
    idx_start(1, 1)

    def pair_step(g, carry):
        for b in range(2):
            o = 1 - b
            c = 2 * g + b
            if b == 0:
                idx_wait(o)
                gather_start(o)
            else:
                @pl.when(g < N_CHUNKS // 2 - 1)
                def _():
                    idx_wait(o)
                    gather_start(o)
            gather_wait(b)

            @pl.when(g >= 1)
            def _():
                store_wait(b)

            add_enc(c, b)
            store_start(c, b)

            @pl.when(g < N_CHUNKS // 2 - 1)
            def _():
                idx_start(c + 2, b)
        return carry

    lax.fori_loop(0, N_CHUNKS // 2, pair_step, 0)
    store_wait(0)
    store_wait(1)


@jax.jit
def kernel(x, table):
    seq_len = x.shape[1]
    enc = _make_encoding(seq_len)
    enc2 = jnp.concatenate([enc, enc], axis=0)
    idx = x.reshape(-1)

    mesh = plsc.VectorSubcoreMesh(core_axis_name="c", subcore_axis_name="s")
    run = pl.kernel(
        _emb_body,
        out_type=jax.ShapeDtypeStruct((N_ROWS, D_MODEL), jnp.float32),
        mesh=mesh,
        scratch_types=[
            pltpu.VMEM((2 * SEQ_LEN, D_MODEL), jnp.float32),
            pltpu.VMEM((2, CHUNK), jnp.int32),
            pltpu.VMEM((CHUNK, D_MODEL), jnp.float32),
            pltpu.VMEM((CHUNK, D_MODEL), jnp.float32),
            pltpu.VMEM((CHUNK, D_MODEL), jnp.float32),
            pltpu.VMEM((CHUNK, D_MODEL), jnp.float32),
            pltpu.SemaphoreType.DMA,
            pltpu.SemaphoreType.DMA,
            pltpu.SemaphoreType.DMA,
            pltpu.SemaphoreType.DMA,
            pltpu.SemaphoreType.DMA,
            pltpu.SemaphoreType.DMA,
        ],
        compiler_params=pltpu.CompilerParams(use_tc_tiling_on_sc=False),
    )
    out = run(table, idx, enc2)
    return out.reshape(x.shape[0], seq_len, D_MODEL)


# speedup vs baseline: 3.9298x; 1.0525x over previous
import jax
import jax.numpy as jnp
from jax import lax
from jax.experimental import pallas as pl
from jax.experimental.pallas import tpu as pltpu
from jax.experimental.pallas import tpu_sc as plsc

D_MODEL = 64
SEQ_LEN = 200
BATCH = 4096
NUM_WORKERS = 32
N_ROWS = BATCH * SEQ_LEN
BROWS_PER_W = BATCH // NUM_WORKERS
G0 = 128
G1 = SEQ_LEN - G0


def _make_encoding(seq_len: int) -> jax.Array:
    pos = jnp.arange(seq_len, dtype=jnp.float32)[:, None]
    _2i = jnp.arange(0, D_MODEL, 2, dtype=jnp.float32)
    enc = jnp.zeros((seq_len, D_MODEL), dtype=jnp.float32)
    enc = enc.at[:, 0::2].set(jnp.sin(pos / (10000.0 ** (_2i / D_MODEL))))
    enc = enc.at[:, 1::2].set(jnp.cos(pos / (10000.0 ** (_2i / D_MODEL))))
    return enc


def _emb_body(table_hbm, idx_hbm, enc_hbm, out_hbm,
              enc_v, idx0, idx1, rin0, rin1, rout0, rout1,
              sg0, sg1, so0, so1, si0, si1):
    wid = lax.axis_index("s") * 2 + lax.axis_index("c")
    base_b = wid * BROWS_PER_W
    idx = (idx0, idx1)
    rin = (rin0, rin1)
    rout = (rout0, rout1)
    sg = (sg0, sg1)
    so = (so0, so1)
    si = (si0, si1)

    def idx_start(c, b):
        pltpu.make_async_copy(
            idx_hbm.at[pl.ds((base_b + c) * SEQ_LEN, SEQ_LEN)], idx[b], si[b]
        ).start()

    def idx_wait(b):
        pltpu.make_async_copy(
            idx_hbm.at[pl.ds(0, SEQ_LEN)], idx[b], si[b]
        ).wait()

    def gather_start(b):
        pltpu.make_async_copy(
            table_hbm.at[idx[b].at[pl.ds(0, G0)]], rin[b].at[pl.ds(0, G0)],
            sg[b]).start()
        pltpu.make_async_copy(
            table_hbm.at[idx[b].at[pl.ds(G0, G1)]], rin[b].at[pl.ds(G0, G1)],
            sg[b]).start()

    def gather_wait(b):
        pltpu.make_async_copy(
            table_hbm.at[idx[b].at[pl.ds(0, G0)]], rin[b].at[pl.ds(0, G0)],
            sg[b]).wait()
        pltpu.make_async_copy(
            table_hbm.at[idx[b].at[pl.ds(G0, G1)]], rin[b].at[pl.ds(G0, G1)],
            sg[b]).wait()

    def store_start(c, b):
        pltpu.make_async_copy(
            rout[b], out_hbm.at[pl.ds(base_b + c, 1)], so[b]
        ).start()

    def store_wait(b):
        pltpu.make_async_copy(
            rout[b], out_hbm.at[pl.ds(0, 1)], so[b]
        ).wait()

    def add_enc(b):
        src = rin[b]
        dst = rout[b]

        @plsc.parallel_loop(0, SEQ_LEN, 1, unroll=4)
        def _(j):
            for d in range(D_MODEL // 16):
                sl = pl.ds(d * 16, 16)
                dst[0, j, sl] = src[j, sl] + enc_v[j, sl]

    pltpu.sync_copy(enc_hbm, enc_v)

    pltpu.sync_copy(idx_hbm.at[pl.ds(base_b * SEQ_LEN, SEQ_LEN)], idx0)
    gather_start(0)
    idx_start(1, 1)

    def pair_step(g, carry):
        for b in range(2):
            o = 1 - b
            c = 2 * g + b
            if b == 0:
                idx_wait(o)
                gather_start(o)
            else:
                @pl.when(g < BROWS_PER_W // 2 - 1)
                def _():
                    idx_wait(o)
                    gather_start(o)
            gather_wait(b)

            @pl.when(g >= 1)
            def _():
                store_wait(b)

            add_enc(b)
            store_start(c, b)

            @pl.when(g < BROWS_PER_W // 2 - 1)
            def _():
                idx_start(c + 2, b)
        return carry

    lax.fori_loop(0, BROWS_PER_W // 2, pair_step, 0)
    store_wait(0)
    store_wait(1)


@jax.jit
def kernel(x, table):
    seq_len = x.shape[1]
    enc = _make_encoding(seq_len)
    idx = x.reshape(-1)

    mesh = plsc.VectorSubcoreMesh(core_axis_name="c", subcore_axis_name="s")
    run = pl.kernel(
        _emb_body,
        out_type=jax.ShapeDtypeStruct((BATCH, SEQ_LEN, D_MODEL), jnp.float32),
        mesh=mesh,
        scratch_types=[
            pltpu.VMEM((SEQ_LEN, D_MODEL), jnp.float32),
            pltpu.VMEM((SEQ_LEN,), jnp.int32),
            pltpu.VMEM((SEQ_LEN,), jnp.int32),
            pltpu.VMEM((SEQ_LEN, D_MODEL), jnp.float32),
            pltpu.VMEM((SEQ_LEN, D_MODEL), jnp.float32),
            pltpu.VMEM((1, SEQ_LEN, D_MODEL), jnp.float32),
            pltpu.VMEM((1, SEQ_LEN, D_MODEL), jnp.float32),
            pltpu.SemaphoreType.DMA,
            pltpu.SemaphoreType.DMA,
            pltpu.SemaphoreType.DMA,
            pltpu.SemaphoreType.DMA,
            pltpu.SemaphoreType.DMA,
            pltpu.SemaphoreType.DMA,
        ],
        compiler_params=pltpu.CompilerParams(use_tc_tiling_on_sc=False),
    )
    return run(table, idx, enc)


# speedup vs baseline: 4.6736x; 1.1893x over previous
import jax
import jax.numpy as jnp
from jax import lax
from jax.experimental import pallas as pl
from jax.experimental.pallas import tpu as pltpu
from jax.experimental.pallas import tpu_sc as plsc

D_MODEL = 64
DPAD = 128
SEQ_LEN = 200
BATCH = 4096
NUM_WORKERS = 32
N_ROWS = BATCH * SEQ_LEN
BROWS_PER_W = BATCH // NUM_WORKERS
G0 = 128
G1 = SEQ_LEN - G0


def _make_encoding(seq_len: int) -> jax.Array:
    pos = jnp.arange(seq_len, dtype=jnp.float32)[:, None]
    _2i = jnp.arange(0, D_MODEL, 2, dtype=jnp.float32)
    enc = jnp.zeros((seq_len, D_MODEL), dtype=jnp.float32)
    enc = enc.at[:, 0::2].set(jnp.sin(pos / (10000.0 ** (_2i / D_MODEL))))
    enc = enc.at[:, 1::2].set(jnp.cos(pos / (10000.0 ** (_2i / D_MODEL))))
    return enc


def _emb_body(table_hbm, idx_hbm, enc_hbm, out_hbm,
              enc_v, idx0, idx1, rin0, rin1, rout0, rout1,
              sg0, sg1, so0, so1, si0, si1):
    wid = lax.axis_index("s") * 2 + lax.axis_index("c")
    base_b = wid * BROWS_PER_W
    idx = (idx0, idx1)
    rin = (rin0, rin1)
    rout = (rout0, rout1)
    sg = (sg0, sg1)
    so = (so0, so1)
    si = (si0, si1)

    def idx_start(c, b):
        pltpu.make_async_copy(
            idx_hbm.at[pl.ds((base_b + c) * SEQ_LEN, SEQ_LEN)], idx[b], si[b]
        ).start()

    def idx_wait(b):
        pltpu.make_async_copy(
            idx_hbm.at[pl.ds(0, SEQ_LEN)], idx[b], si[b]
        ).wait()

    def gather_start(b):
        pltpu.make_async_copy(
            table_hbm.at[idx[b].at[pl.ds(0, G0)]], rin[b].at[pl.ds(0, G0)],
            sg[b]).start()
        pltpu.make_async_copy(
            table_hbm.at[idx[b].at[pl.ds(G0, G1)]], rin[b].at[pl.ds(G0, G1)],
            sg[b]).start()

    def gather_wait(b):
        pltpu.make_async_copy(
            table_hbm.at[idx[b].at[pl.ds(0, G0)]], rin[b].at[pl.ds(0, G0)],
            sg[b]).wait()
        pltpu.make_async_copy(
            table_hbm.at[idx[b].at[pl.ds(G0, G1)]], rin[b].at[pl.ds(G0, G1)],
            sg[b]).wait()

    def store_start(c, b):
        pltpu.make_async_copy(
            rout[b], out_hbm.at[pl.ds(base_b + c, 1)], so[b]
        ).start()

    def store_wait(b):
        pltpu.make_async_copy(
            rout[b], out_hbm.at[pl.ds(0, 1)], so[b]
        ).wait()

    def add_enc(b):
        src = rin[b]
        dst = rout[b]

        @plsc.parallel_loop(0, SEQ_LEN, 1, unroll=4)
        def _(j):
            for d in range(D_MODEL // 16):
                sl = pl.ds(d * 16, 16)
                dst[0, j, sl] = src[j, sl] + enc_v[j, sl]

    pltpu.sync_copy(enc_hbm, enc_v)

    pltpu.sync_copy(idx_hbm.at[pl.ds(base_b * SEQ_LEN, SEQ_LEN)], idx0)
    gather_start(0)
    idx_start(1, 1)

    def pair_step(g, carry):
        for b in range(2):
            o = 1 - b
            c = 2 * g + b
            if b == 0:
                idx_wait(o)
                gather_start(o)
            else:
                @pl.when(g < BROWS_PER_W // 2 - 1)
                def _():
                    idx_wait(o)
                    gather_start(o)
            gather_wait(b)

            @pl.when(g >= 1)
            def _():
                store_wait(b)

            add_enc(b)
            store_start(c, b)

            @pl.when(g < BROWS_PER_W // 2 - 1)
            def _():
                idx_start(c + 2, b)
        return carry

    lax.fori_loop(0, BROWS_PER_W // 2, pair_step, 0)
    store_wait(0)
    store_wait(1)


@jax.jit
def kernel(x, table):
    seq_len = x.shape[1]
    enc = _make_encoding(seq_len)
    idx = x.reshape(-1)
    table_p = jnp.pad(table, ((0, 0), (0, DPAD - D_MODEL)))

    mesh = plsc.VectorSubcoreMesh(core_axis_name="c", subcore_axis_name="s")
    run = pl.kernel(
        _emb_body,
        out_type=jax.ShapeDtypeStruct((BATCH, SEQ_LEN, D_MODEL), jnp.float32),
        mesh=mesh,
        scratch_types=[
            pltpu.VMEM((SEQ_LEN, D_MODEL), jnp.float32),
            pltpu.VMEM((SEQ_LEN,), jnp.int32),
            pltpu.VMEM((SEQ_LEN,), jnp.int32),
            pltpu.VMEM((SEQ_LEN, DPAD), jnp.float32),
            pltpu.VMEM((SEQ_LEN, DPAD), jnp.float32),
            pltpu.VMEM((1, SEQ_LEN, D_MODEL), jnp.float32),
            pltpu.VMEM((1, SEQ_LEN, D_MODEL), jnp.float32),
            pltpu.SemaphoreType.DMA,
            pltpu.SemaphoreType.DMA,
            pltpu.SemaphoreType.DMA,
            pltpu.SemaphoreType.DMA,
            pltpu.SemaphoreType.DMA,
            pltpu.SemaphoreType.DMA,
        ],
        compiler_params=pltpu.CompilerParams(use_tc_tiling_on_sc=True),
    )
    return run(table_p, idx, enc)
